# baseline (speedup 1.0000x reference)
import numpy as np

import jax
import jax.numpy as jnp
from jax import lax
from jax.experimental import pallas as pl
from jax.experimental.pallas import tpu as pltpu
from jax.experimental.pallas import tpu_sc as plsc

NC = 2
NS = 16
L = 16
NW = NC * NS

N0 = 128
N1 = 32
COLS = 8192
SR = 8
HALF = COLS // 2
TILES = HALF // 128
KPT = 128 // L

N_TC = 80
N_SC = N0 - N_TC
SC_ROWS = N_SC * N1
SPW = SC_ROWS // (SR * NW)
RPW = SPW * SR

BIG = np.int32(COLS)


def _sc_body(x_hbm, out_hbm, buf_a, buf_b, res, sem_a, sem_b):
    c = lax.axis_index("c")
    s = lax.axis_index("s")
    wid = s * NC + c

    lane = lax.iota(jnp.int32, L)

    def src(strip, half):
        n = N_TC + strip // (N1 // SR)
        h0 = (strip % (N1 // SR)) * SR
        return x_hbm.at[n, pl.ds(h0, SR), pl.ds(half * HALF, HALF)]

    RG = 4

    def process_half(buf, col_base, vmins, vidxs):
        vmins, vidxs = list(vmins), list(vidxs)
        for r0 in range(0, SR, RG):
            def step(t, carry, r0=r0):
                vm, vi = list(carry[0]), list(carry[1])
                for k in range(KPT):
                    col = t * 128 + k * L
                    idx = col_base + col + lane
                    for j in range(RG):
                        v = buf[r0 + j, pl.ds(col, L)]
                        m = v < vm[j]
                        vm[j] = jnp.where(m, v, vm[j])
                        vi[j] = jnp.where(m, idx, vi[j])
                return tuple(vm), tuple(vi)

            vm, vi = lax.fori_loop(
                0, TILES, step,
                (tuple(vmins[r0:r0 + RG]), tuple(vidxs[r0:r0 + RG])),
            )
            vmins[r0:r0 + RG] = list(vm)
            vidxs[r0:r0 + RG] = list(vi)
        return tuple(vmins), tuple(vidxs)

    dnums = lax.GatherDimensionNumbers(
        offset_dims=(), collapsed_slice_dims=(0,), start_index_map=(0,)
    )

    def shuffle(v, perm):
        return lax.gather(
            v, perm[:, None], dnums, slice_sizes=(1,),
            mode=lax.GatherScatterMode.PROMISE_IN_BOUNDS,
        )

    perms = [lane ^ (1 << d) for d in range(4)]

    def epilogue(vmins, vidxs, acc, slot0):
        for r in range(SR):
            vm = vmins[r]
            for p in perms:
                vm = jnp.minimum(vm, shuffle(vm, p))
            cand = jnp.where(vmins[r] == vm, vidxs[r], jnp.full((L,), BIG))
            for p in perms:
                cand = jnp.minimum(cand, shuffle(cand, p))
            acc = jnp.where(lane == (slot0 + r), cand, acc)
        return acc

    strip0 = wid * SPW
    pltpu.async_copy(src(strip0, 0), buf_a, sem_a)

    def strip_body(i, acc):
        strip = strip0 + i
        cp_b = pltpu.async_copy(src(strip, 1), buf_b, sem_b)
        pltpu.make_async_copy(src(strip, 0), buf_a, sem_a).wait()

        vmins = tuple(jnp.full((L,), jnp.inf, jnp.float32) for _ in range(SR))
        vidxs = tuple(jnp.zeros((L,), jnp.int32) for _ in range(SR))
        vmins, vidxs = process_half(buf_a, 0, vmins, vidxs)

        @pl.when(i + 1 < SPW)
        def _():
            pltpu.async_copy(src(strip + 1, 0), buf_a, sem_a)

        cp_b.wait()
        vmins, vidxs = process_half(buf_b, HALF, vmins, vidxs)

        acc = epilogue(vmins, vidxs, acc, (i % 2) * SR)

        @pl.when(i % 2 == 1)
        def _():
            res[pl.ds((i - 1) * SR, L)] = acc

        return jnp.where(i % 2 == 1, jnp.zeros((L,), jnp.int32), acc)

    lax.fori_loop(0, SPW, strip_body, jnp.zeros((L,), jnp.int32))

    pltpu.sync_copy(res, out_hbm.at[pl.ds(wid * RPW, RPW)])


def _sc_call(x):
    call = pl.kernel(
        _sc_body,
        out_type=jax.ShapeDtypeStruct((SC_ROWS,), jnp.int32),
        mesh=plsc.VectorSubcoreMesh(core_axis_name="c", subcore_axis_name="s"),
        scratch_types=[
            pltpu.VMEM((SR, HALF), jnp.float32),
            pltpu.VMEM((SR, HALF), jnp.float32),
            pltpu.VMEM((RPW,), jnp.int32),
            pltpu.SemaphoreType.DMA,
            pltpu.SemaphoreType.DMA,
        ],
        compiler_params=pltpu.CompilerParams(use_tc_tiling_on_sc=True),
    )
    return call(x)


BN = 16


def _tc_block(x_ref, o_ref):
    xb = x_ref[...]
    m = jnp.min(xb, axis=2, keepdims=True)
    iota = lax.broadcasted_iota(jnp.int32, xb.shape, 2)
    cand = jnp.where(xb == m, iota, COLS)
    o_ref[...] = jnp.min(cand, axis=2).astype(jnp.int32)


def _tc_call(x):
    return pl.pallas_call(
        _tc_block,
        grid=(N_TC // BN,),
        in_specs=[pl.BlockSpec((BN, N1, COLS), lambda i: (i, 0, 0))],
        out_specs=pl.BlockSpec((BN, N1), lambda i: (i, 0)),
        out_shape=jax.ShapeDtypeStruct((N_TC, N1), jnp.int32),
    )(x)


@jax.jit
def _argmin_hybrid(x):
    out_sc = _sc_call(x)
    out_tc = _tc_call(x)
    return jnp.concatenate([out_tc, out_sc.reshape(N_SC, N1)], axis=0)


def kernel(x):
    return _argmin_hybrid(x)
